# baseline (speedup 1.0000x reference)
import functools

import jax
import jax.numpy as jnp
from jax import lax
from jax.experimental import pallas as pl
from jax.experimental.pallas import tpu as pltpu
from jax.experimental.pallas import tpu_sc as plsc

K = 2048
NC, NS, L = 2, 16, 16
NW = NC * NS
CHUNK = 1024


def _sc_hist(P, C):
    PW = P // NW
    NCH = PW // CHUNK
    GRP = CHUNK // L

    mesh = plsc.VectorSubcoreMesh(core_axis_name="c", subcore_axis_name="s")

    @functools.partial(
        pl.kernel,
        out_type=(
            jax.ShapeDtypeStruct((NW, C, K), jnp.int32),
            jax.ShapeDtypeStruct((NW, C, K), jnp.int32),
        ),
        mesh=mesh,
        scratch_types=[
            pltpu.VMEM((CHUNK, C), jnp.float32),
            pltpu.VMEM((CHUNK, C), jnp.float32),
            pltpu.VMEM((CHUNK,), jnp.int32),
            pltpu.VMEM((CHUNK,), jnp.int32),
            pltpu.VMEM((C, K), jnp.int32),
            pltpu.VMEM((C, K), jnp.int32),
            pltpu.SemaphoreType.DMA,
            pltpu.SemaphoreType.DMA,
            pltpu.SemaphoreType.DMA,
            pltpu.SemaphoreType.DMA,
        ],
        compiler_params=pltpu.CompilerParams(
            needs_layout_passes=False, use_tc_tiling_on_sc=False
        ),
    )
    def kern(probas_hbm, labels_hbm, hall_hbm, hfg_hbm,
             pbuf0, pbuf1, lbuf0, lbuf1, ha, hf, ps0, ps1, ls0, ls1):
        wid = lax.axis_index("s") * NC + lax.axis_index("c")
        zeros16 = jnp.zeros((L,), jnp.int32)

        def zbody(j, _):
            for c in range(C):
                ha[c, pl.ds(j * L, L)] = zeros16
                hf[c, pl.ds(j * L, L)] = zeros16
            return 0

        lax.fori_loop(0, K // L, zbody, 0)

        lanes = lax.iota(jnp.int32, L)
        ones16 = jnp.ones((L,), jnp.int32)
        one_f = jnp.full((L,), 1.0, jnp.float32)
        m11 = jnp.full((L,), K - 1, jnp.int32)

        def start(t, pb, lb, psem, lsem):
            pt0 = wid * PW + t * CHUNK
            pltpu.async_copy(probas_hbm.at[pl.ds(pt0, CHUNK)], pb, psem)
            pltpu.async_copy(labels_hbm.at[0, pl.ds(pt0, CHUNK)], lb, lsem)

        def wait(pb, lb, psem, lsem):
            pltpu.make_async_copy(probas_hbm.at[pl.ds(0, CHUNK)], pb, psem).wait()
            pltpu.make_async_copy(labels_hbm.at[0, pl.ds(0, CHUNK)], lb, lsem).wait()

        def bucket(p):
            return jax.lax.shift_right_logical(
                plsc.bitcast(p + one_f, jnp.int32), 12) & m11

        def process(pb, lb):
            def grp_body(g, _):
                ridx = g * L + lanes
                lbl = lb[pl.ds(g * L, L)]
                pfg = plsc.load_gather(pb, [ridx, lbl])
                plsc.addupdate_scatter(hf, [lbl, bucket(pfg)], ones16)
                for c in range(C):
                    p = plsc.load_gather(pb, [ridx, jnp.full((L,), c, jnp.int32)])
                    plsc.addupdate_scatter(
                        ha, [jnp.full((L,), c, jnp.int32), bucket(p)], ones16
                    )
                return 0

            lax.fori_loop(0, GRP, grp_body, 0)

        start(0, pbuf0, lbuf0, ps0, ls0)

        def pair_body(i, _):
            t0 = 2 * i
            start(t0 + 1, pbuf1, lbuf1, ps1, ls1)
            wait(pbuf0, lbuf0, ps0, ls0)
            process(pbuf0, lbuf0)

            @pl.when(i < NCH // 2 - 1)
            def _():
                start(t0 + 2, pbuf0, lbuf0, ps0, ls0)

            wait(pbuf1, lbuf1, ps1, ls1)
            process(pbuf1, lbuf1)
            return 0

        lax.fori_loop(0, NCH // 2, pair_body, 0)
        pltpu.sync_copy(ha, hall_hbm.at[wid])
        pltpu.sync_copy(hf, hfg_hbm.at[wid])

    return kern


def _tc_phase2(C):
    TK = 256

    def kern(hall_ref, hfg_ref, out_ref):
        hall = jnp.sum(hall_ref[...].astype(jnp.float32), axis=0)
        hfg = jnp.sum(hfg_ref[...].astype(jnp.float32), axis=0)
        hbg = hall - hfg
        gts = jnp.sum(hfg, axis=1, keepdims=True)
        jrow = lax.broadcasted_iota(jnp.int32, (K, TK), 0)
        kcol = lax.broadcasted_iota(jnp.int32, (K, TK), 1)
        total = jnp.float32(0.0)
        f0sum = jnp.float32(0.0)
        for tile in range(K // TK):
            k0 = tile * TK
            m1 = jnp.where(jrow + kcol + k0 < K, 1.0, 0.0)
            m2 = jnp.where(jrow >= kcol + k0, 1.0, 0.0)
            a = jnp.dot(hfg, m1, preferred_element_type=jnp.float32)
            b = jnp.dot(hbg, m2, preferred_element_type=jnp.float32)
            denom = gts + b
            F = jnp.where(
                denom > 0.0,
                1.0 - (gts - a) / jnp.where(denom > 0.0, denom, 1.0),
                0.0,
            )
            total = total + jnp.sum(F)
            if tile == 0:
                f0sum = jnp.sum(F[:, 0:1])
        out_ref[0, 0] = (total - 0.5 * f0sum) / jnp.float32(C * K)

    return kern


def kernel(probas, labels):
    P, C = probas.shape
    hall, hfg = _sc_hist(P, C)(probas, labels)
    out = pl.pallas_call(
        _tc_phase2(C),
        out_shape=jax.ShapeDtypeStruct((1, 1), jnp.float32),
        out_specs=pl.BlockSpec(memory_space=pltpu.SMEM),
    )(hall, hfg)
    return out[0, 0]


# speedup vs baseline: 3.3367x; 3.3367x over previous
import functools

import jax
import jax.numpy as jnp
from jax import lax
from jax.experimental import pallas as pl
from jax.experimental.pallas import tpu as pltpu
from jax.experimental.pallas import tpu_sc as plsc

K = 2048
NC, NS, L = 2, 16, 16
NW = NC * NS
BW = 65536
CHUNK = 16384


def _bucket(p):
    return jax.lax.shift_right_logical(
        jax.lax.bitcast_convert_type(p + 1.0, jnp.int32), 12
    ) & (K - 1)


def _tc_bucketize(P, C):
    def kern(pt_ref, *out_refs):
        idx = _bucket(pt_ref[...])
        rows = lax.broadcasted_iota(jnp.int32, (C, BW), 0)
        idx = idx + rows * K
        for c in range(C):
            out_refs[c][...] = idx[c : c + 1, :].reshape(BW)

    return pl.pallas_call(
        kern,
        grid=(P // BW,),
        in_specs=[pl.BlockSpec((C, BW), lambda i: (0, i))],
        out_specs=[pl.BlockSpec((BW,), lambda i: (i,)) for _ in range(C)],
        out_shape=[jax.ShapeDtypeStruct((P,), jnp.int32) for _ in range(C)],
    )


def _tc_fg_bucketize(P, C):
    def kern(pt_ref, lbl_ref, out_ref):
        pt = pt_ref[...]
        lbl = lbl_ref[...]
        rows = lax.broadcasted_iota(jnp.int32, (C, BW), 0)
        pfg = jnp.sum(jnp.where(rows == lbl, pt, 0.0), axis=0, keepdims=True)
        idx = C * K + lbl * K + _bucket(pfg)
        out_ref[...] = idx.reshape(BW)

    return pl.pallas_call(
        kern,
        grid=(P // BW,),
        in_specs=[
            pl.BlockSpec((C, BW), lambda i: (0, i)),
            pl.BlockSpec((1, BW), lambda i: (0, i)),
        ],
        out_specs=pl.BlockSpec((BW,), lambda i: (i,)),
        out_shape=jax.ShapeDtypeStruct((P,), jnp.int32),
    )


def _sc_scatter(P, C):
    H = 2 * C * K
    PW = P // NW
    NCH = PW // CHUNK
    mesh = plsc.VectorSubcoreMesh(core_axis_name="c", subcore_axis_name="s")

    @functools.partial(
        pl.kernel,
        out_type=jax.ShapeDtypeStruct((NW, H), jnp.int32),
        mesh=mesh,
        scratch_types=[
            pltpu.VMEM((CHUNK,), jnp.int32),
            pltpu.VMEM((CHUNK,), jnp.int32),
            pltpu.VMEM((H,), jnp.int32),
            pltpu.SemaphoreType.DMA,
            pltpu.SemaphoreType.DMA,
        ],
        compiler_params=pltpu.CompilerParams(
            needs_layout_passes=False, use_tc_tiling_on_sc=False
        ),
    )
    def kern(*refs):
        srcs = refs[: C + 1]
        hist_hbm = refs[C + 1]
        buf0, buf1, hist, s0, s1 = refs[C + 2 :]
        bufs = (buf0, buf1)
        sems = (s0, s1)
        wid = lax.axis_index("s") * NC + lax.axis_index("c")
        zeros16 = jnp.zeros((L,), jnp.int32)

        def zbody(j, _):
            hist[pl.ds(j * L, L)] = zeros16
            return 0

        lax.fori_loop(0, H // L, zbody, 0)

        ones16 = jnp.ones((L,), jnp.int32)
        tasks = [(src, j) for src in srcs for j in range(NCH)]

        def start(t, b):
            src, j = tasks[t]
            pltpu.async_copy(
                src.at[pl.ds(wid * PW + j * CHUNK, CHUNK)], bufs[b], sems[b]
            )

        def wait(b):
            pltpu.make_async_copy(
                srcs[0].at[pl.ds(0, CHUNK)], bufs[b], sems[b]
            ).wait()

        def process(b):
            buf = bufs[b]

            def ubody(u, _):
                idx = buf[pl.ds(u * L, L)]
                plsc.addupdate_scatter(hist, [idx], ones16)
                return 0

            lax.fori_loop(0, CHUNK // L, ubody, 0)

        start(0, 0)
        for t in range(len(tasks)):
            if t + 1 < len(tasks):
                start(t + 1, (t + 1) % 2)
            wait(t % 2)
            process(t % 2)
        pltpu.sync_copy(hist, hist_hbm.at[wid])

    return kern


def _tc_phase2(C):
    TK = 256

    def kern(h_ref, out_ref):
        h = jnp.sum(h_ref[...].astype(jnp.float32), axis=0)
        hall = h[0]
        hfg = h[1]
        hbg = hall - hfg
        gts = jnp.sum(hfg, axis=1, keepdims=True)
        jrow = lax.broadcasted_iota(jnp.int32, (K, TK), 0)
        kcol = lax.broadcasted_iota(jnp.int32, (K, TK), 1)
        total = jnp.float32(0.0)
        f0sum = jnp.float32(0.0)
        for tile in range(K // TK):
            k0 = tile * TK
            m1 = jnp.where(jrow + kcol + k0 < K, 1.0, 0.0)
            m2 = jnp.where(jrow >= kcol + k0, 1.0, 0.0)
            a = jnp.dot(hfg, m1, preferred_element_type=jnp.float32)
            b = jnp.dot(hbg, m2, preferred_element_type=jnp.float32)
            denom = gts + b
            F = jnp.where(
                denom > 0.0,
                1.0 - (gts - a) / jnp.where(denom > 0.0, denom, 1.0),
                0.0,
            )
            total = total + jnp.sum(F)
            if tile == 0:
                f0sum = jnp.sum(F[:, 0:1])
        out_ref[0, 0] = (total - 0.5 * f0sum) / jnp.float32(C * K)

    return kern


def kernel(probas, labels):
    P, C = probas.shape
    probasT = probas.T
    hall_idx = _tc_bucketize(P, C)(probasT)
    fg_idx = _tc_fg_bucketize(P, C)(probasT, labels)
    hist = _sc_scatter(P, C)(*hall_idx, fg_idx)
    hist4 = hist.reshape(NW, 2, C, K)
    out = pl.pallas_call(
        _tc_phase2(C),
        out_shape=jax.ShapeDtypeStruct((1, 1), jnp.float32),
        out_specs=pl.BlockSpec(memory_space=pltpu.SMEM),
    )(hist4)
    return out[0, 0]


# speedup vs baseline: 7.5057x; 2.2495x over previous
import functools

import jax
import jax.numpy as jnp
from jax import lax
from jax.experimental import pallas as pl
from jax.experimental.pallas import tpu as pltpu
from jax.experimental.pallas import tpu_sc as plsc

K = 2048
NC, NS, L = 2, 16, 16
NW = NC * NS
BW = 65536
CHUNK = 16384


def _bucket(p):
    return jax.lax.shift_right_logical(
        jax.lax.bitcast_convert_type(p + 1.0, jnp.int32), 12
    ) & (K - 1)


def _tc_bucketize(P, C):
    def kern(pt_ref, lbl_ref, *out_refs):
        pt = pt_ref[...]
        rows = lax.broadcasted_iota(jnp.int32, (C, BW), 0)
        idx = _bucket(pt) + rows * K
        for c in range(C):
            out_refs[c][...] = idx[c : c + 1, :].reshape(BW)
        lbl = lbl_ref[...]
        pfg = jnp.sum(jnp.where(rows == lbl, pt, 0.0), axis=0, keepdims=True)
        out_refs[C][...] = (C * K + lbl * K + _bucket(pfg)).reshape(BW)

    return pl.pallas_call(
        kern,
        grid=(P // BW,),
        in_specs=[
            pl.BlockSpec((C, BW), lambda i: (0, i)),
            pl.BlockSpec((1, BW), lambda i: (0, i)),
        ],
        out_specs=[pl.BlockSpec((BW,), lambda i: (i,)) for _ in range(C + 1)],
        out_shape=[jax.ShapeDtypeStruct((P,), jnp.int32) for _ in range(C + 1)],
    )


def _sc_scatter(P, C):
    H = 2 * C * K
    PW = P // NW
    NCH = PW // CHUNK
    mesh = plsc.VectorSubcoreMesh(core_axis_name="c", subcore_axis_name="s")

    @functools.partial(
        pl.kernel,
        out_type=jax.ShapeDtypeStruct((NW, H), jnp.int32),
        mesh=mesh,
        scratch_types=[
            pltpu.VMEM((CHUNK,), jnp.int32),
            pltpu.VMEM((CHUNK,), jnp.int32),
            pltpu.VMEM((H,), jnp.int32),
            pltpu.SemaphoreType.DMA,
            pltpu.SemaphoreType.DMA,
        ],
        compiler_params=pltpu.CompilerParams(
            needs_layout_passes=False, use_tc_tiling_on_sc=False
        ),
    )
    def kern(*refs):
        srcs = refs[: C + 1]
        hist_hbm = refs[C + 1]
        buf0, buf1, hist, s0, s1 = refs[C + 2 :]
        bufs = (buf0, buf1)
        sems = (s0, s1)
        wid = lax.axis_index("s") * NC + lax.axis_index("c")
        zeros16 = jnp.zeros((L,), jnp.int32)

        def zbody(j, _):
            hist[pl.ds(j * L, L)] = zeros16
            return 0

        lax.fori_loop(0, H // L, zbody, 0)

        ones16 = jnp.ones((L,), jnp.int32)
        tasks = [(src, j) for src in srcs for j in range(NCH)]

        def start(t, b):
            src, j = tasks[t]
            pltpu.async_copy(
                src.at[pl.ds(wid * PW + j * CHUNK, CHUNK)], bufs[b], sems[b]
            )

        def wait(b):
            pltpu.make_async_copy(
                srcs[0].at[pl.ds(0, CHUNK)], bufs[b], sems[b]
            ).wait()

        def process(b):
            buf = bufs[b]

            @plsc.parallel_loop(0, CHUNK // L, unroll=8)
            def ubody(u):
                idx = buf[pl.ds(u * L, L)]
                plsc.addupdate_scatter(hist, [idx], ones16)

        start(0, 0)
        for t in range(len(tasks)):
            if t + 1 < len(tasks):
                start(t + 1, (t + 1) % 2)
            wait(t % 2)
            process(t % 2)
        pltpu.sync_copy(hist, hist_hbm.at[wid])

    return kern


def _tc_phase2(C):
    TK = 256

    def kern(h_ref, out_ref):
        h = jnp.sum(h_ref[...].astype(jnp.float32), axis=0)
        hall = h[0]
        hfg = h[1]
        hbg = hall - hfg
        gts = jnp.sum(hfg, axis=1, keepdims=True)
        jrow = lax.broadcasted_iota(jnp.int32, (K, TK), 0)
        kcol = lax.broadcasted_iota(jnp.int32, (K, TK), 1)
        total = jnp.float32(0.0)
        f0sum = jnp.float32(0.0)
        for tile in range(K // TK):
            k0 = tile * TK
            m1 = jnp.where(jrow + kcol + k0 < K, 1.0, 0.0)
            m2 = jnp.where(jrow >= kcol + k0, 1.0, 0.0)
            a = jnp.dot(hfg, m1, preferred_element_type=jnp.float32)
            b = jnp.dot(hbg, m2, preferred_element_type=jnp.float32)
            denom = gts + b
            F = jnp.where(
                denom > 0.0,
                1.0 - (gts - a) / jnp.where(denom > 0.0, denom, 1.0),
                0.0,
            )
            total = total + jnp.sum(F)
            if tile == 0:
                f0sum = jnp.sum(F[:, 0:1])
        out_ref[0, 0] = (total - 0.5 * f0sum) / jnp.float32(C * K)

    return kern


def kernel(probas, labels):
    P, C = probas.shape
    probasT = probas.T
    streams = _tc_bucketize(P, C)(probasT, labels)
    hist = _sc_scatter(P, C)(*streams)
    hist4 = hist.reshape(NW, 2, C, K)
    out = pl.pallas_call(
        _tc_phase2(C),
        out_shape=jax.ShapeDtypeStruct((1, 1), jnp.float32),
        out_specs=pl.BlockSpec(memory_space=pltpu.SMEM),
    )(hist4)
    return out[0, 0]


# speedup vs baseline: 7.6517x; 1.0194x over previous
import functools

import jax
import jax.numpy as jnp
from jax import lax
from jax.experimental import pallas as pl
from jax.experimental.pallas import tpu as pltpu
from jax.experimental.pallas import tpu_sc as plsc

K = 2048
NC, NS, L = 2, 16, 16
NW = NC * NS
BW = 65536
CHUNK = 16384


def _bucket(p):
    return jax.lax.shift_right_logical(
        jax.lax.bitcast_convert_type(p + 1.0, jnp.int32), 12
    ) & (K - 1)


def _tc_bucketize(P, C):
    def kern(pt_ref, lbl_ref, *out_refs):
        pt = pt_ref[...]
        rows = lax.broadcasted_iota(jnp.int32, (C, BW), 0)
        idx = _bucket(pt) + rows * K
        for c in range(C):
            out_refs[c][...] = idx[c : c + 1, :].reshape(BW)
        lbl = lbl_ref[...]
        pfg = jnp.sum(jnp.where(rows == lbl, pt, 0.0), axis=0, keepdims=True)
        fg_idx = C * K + lbl * K + _bucket(pfg)
        out_refs[C][...] = fg_idx.reshape(BW)

    return pl.pallas_call(
        kern,
        grid=(P // BW,),
        in_specs=[
            pl.BlockSpec((C, BW), lambda i: (0, i)),
            pl.BlockSpec((1, BW), lambda i: (0, i)),
        ],
        out_specs=[pl.BlockSpec((BW,), lambda i: (i,)) for _ in range(C + 1)],
        out_shape=[jax.ShapeDtypeStruct((P,), jnp.int32) for _ in range(C + 1)],
    )


def _sc_scatter(P, C):
    H = 2 * C * K
    PW = P // NW
    NCH = PW // CHUNK
    mesh = plsc.VectorSubcoreMesh(core_axis_name="c", subcore_axis_name="s")

    @functools.partial(
        pl.kernel,
        out_type=jax.ShapeDtypeStruct((NW, H), jnp.int32),
        mesh=mesh,
        scratch_types=[
            pltpu.VMEM((CHUNK,), jnp.int32),
            pltpu.VMEM((CHUNK,), jnp.int32),
            pltpu.VMEM((H,), jnp.int32),
            pltpu.SemaphoreType.DMA,
            pltpu.SemaphoreType.DMA,
        ],
        compiler_params=pltpu.CompilerParams(
            needs_layout_passes=False, use_tc_tiling_on_sc=False
        ),
    )
    def kern(*refs):
        srcs = refs[: C + 1]
        hist_hbm = refs[C + 1]
        buf0, buf1, hist, s0, s1 = refs[C + 2 :]
        bufs = (buf0, buf1)
        sems = (s0, s1)
        wid = lax.axis_index("s") * NC + lax.axis_index("c")
        zeros16 = jnp.zeros((L,), jnp.int32)

        def zbody(j, _):
            hist[pl.ds(j * L, L)] = zeros16
            return 0

        lax.fori_loop(0, H // L, zbody, 0)

        ones16 = jnp.ones((L,), jnp.int32)
        tasks = [(src, j) for src in srcs for j in range(NCH)]

        def start(t, b):
            src, j = tasks[t]
            pltpu.async_copy(
                src.at[pl.ds(wid * PW + j * CHUNK, CHUNK)], bufs[b], sems[b]
            )

        def wait(b):
            pltpu.make_async_copy(
                srcs[0].at[pl.ds(0, CHUNK)], bufs[b], sems[b]
            ).wait()

        def process(b):
            buf = bufs[b]

            @plsc.parallel_loop(0, CHUNK // L, unroll=16)
            def ubody(u):
                idx = buf[pl.ds(u * L, L)]
                plsc.addupdate_scatter(hist, [idx], ones16)

        start(0, 0)
        for t in range(len(tasks)):
            if t + 1 < len(tasks):
                start(t + 1, (t + 1) % 2)
            wait(t % 2)
            process(t % 2)
        pltpu.sync_copy(hist, hist_hbm.at[wid])

    return kern


def _tc_phase2(C):
    TK = 256

    def kern(h_ref, out_ref):
        h = jnp.sum(h_ref[...].astype(jnp.float32), axis=0)
        h2 = h.reshape(1, 2 * C * K)
        hall = jnp.concatenate(
            [h2[:, c * K : (c + 1) * K] for c in range(C)], axis=0
        )
        hfg = jnp.concatenate(
            [h2[:, (C + c) * K : (C + c + 1) * K] for c in range(C)], axis=0
        )
        hbg = hall - hfg
        gts = jnp.sum(hfg, axis=1, keepdims=True)
        jrow = lax.broadcasted_iota(jnp.int32, (K, TK), 0)
        kcol = lax.broadcasted_iota(jnp.int32, (K, TK), 1)
        total = jnp.float32(0.0)
        f0sum = jnp.float32(0.0)
        for tile in range(K // TK):
            k0 = tile * TK
            m1 = jnp.where(jrow + kcol + k0 < K, 1.0, 0.0)
            m2 = jnp.where(jrow >= kcol + k0, 1.0, 0.0)
            a = jnp.dot(hfg, m1, preferred_element_type=jnp.float32)
            b = jnp.dot(hbg, m2, preferred_element_type=jnp.float32)
            denom = gts + b
            F = jnp.where(
                denom > 0.0,
                1.0 - (gts - a) / jnp.where(denom > 0.0, denom, 1.0),
                0.0,
            )
            total = total + jnp.sum(F)
            if tile == 0:
                f0sum = jnp.sum(F[:, 0:1])
        out_ref[0, 0] = (total - 0.5 * f0sum) / jnp.float32(C * K)

    return kern


def kernel(probas, labels):
    P, C = probas.shape
    probasT = probas.T
    streams = _tc_bucketize(P, C)(probasT, labels)
    hist = _sc_scatter(P, C)(*streams)
    out = pl.pallas_call(
        _tc_phase2(C),
        out_shape=jax.ShapeDtypeStruct((1, 1), jnp.float32),
        out_specs=pl.BlockSpec(memory_space=pltpu.SMEM),
    )(hist)
    return out[0, 0]


# speedup vs baseline: 8.5014x; 1.1110x over previous
import functools

import jax
import jax.numpy as jnp
from jax import lax
from jax.experimental import pallas as pl
from jax.experimental.pallas import tpu as pltpu
from jax.experimental.pallas import tpu_sc as plsc

K = 2048
NC, NS, L = 2, 16, 16
NW = NC * NS
BW = 65536
CHUNK = 16384


def _bucket(p):
    return jax.lax.shift_right_logical(
        jax.lax.bitcast_convert_type(p + 1.0, jnp.int32), 12
    ) & (K - 1)


def _tc_bucketize(P, C):
    NS_OUT = (C + 1) // 2

    def kern(pt_ref, lbl_ref, *out_refs):
        pt = pt_ref[...]
        rows = lax.broadcasted_iota(jnp.int32, (C, BW), 0)
        idx = _bucket(pt) + rows * K
        for s in range(NS_OUT - 1):
            w = idx[2 * s : 2 * s + 1, :] | jax.lax.shift_left(
                idx[2 * s + 1 : 2 * s + 2, :], 16
            )
            out_refs[s][...] = w.reshape(BW)
        lbl = lbl_ref[...]
        pfg = jnp.sum(jnp.where(rows == lbl, pt, 0.0), axis=0, keepdims=True)
        fg_raw = lbl * K + _bucket(pfg)
        w = idx[C - 1 : C, :] | jax.lax.shift_left(fg_raw, 16)
        out_refs[NS_OUT - 1][...] = w.reshape(BW)

    return pl.pallas_call(
        kern,
        grid=(P // BW,),
        in_specs=[
            pl.BlockSpec((C, BW), lambda i: (0, i)),
            pl.BlockSpec((1, BW), lambda i: (0, i)),
        ],
        out_specs=[pl.BlockSpec((BW,), lambda i: (i,)) for _ in range(NS_OUT)],
        out_shape=[jax.ShapeDtypeStruct((P,), jnp.int32) for _ in range(NS_OUT)],
    )


def _sc_scatter(P, C):
    H = 2 * C * K
    PW = P // NW
    NCH = PW // CHUNK
    NS_IN = (C + 1) // 2
    mesh = plsc.VectorSubcoreMesh(core_axis_name="c", subcore_axis_name="s")

    @functools.partial(
        pl.kernel,
        out_type=jax.ShapeDtypeStruct((NW, H), jnp.int32),
        mesh=mesh,
        scratch_types=[
            pltpu.VMEM((CHUNK,), jnp.int32),
            pltpu.VMEM((CHUNK,), jnp.int32),
            pltpu.VMEM((H,), jnp.int32),
            pltpu.SemaphoreType.DMA,
            pltpu.SemaphoreType.DMA,
        ],
        compiler_params=pltpu.CompilerParams(
            needs_layout_passes=False, use_tc_tiling_on_sc=False
        ),
    )
    def kern(*refs):
        srcs = refs[:NS_IN]
        hist_hbm = refs[NS_IN]
        buf0, buf1, hist, s0, s1 = refs[NS_IN + 1 :]
        bufs = (buf0, buf1)
        sems = (s0, s1)
        wid = lax.axis_index("s") * NC + lax.axis_index("c")
        zeros16 = jnp.zeros((L,), jnp.int32)

        def zbody(j, _):
            hist[pl.ds(j * L, L)] = zeros16
            return 0

        lax.fori_loop(0, H // L, zbody, 0)

        ones16 = jnp.ones((L,), jnp.int32)
        lomask = jnp.full((L,), 0xFFFF, jnp.int32)
        tasks = [(s, j) for s in range(NS_IN) for j in range(NCH)]

        def start(t, b):
            s, j = tasks[t]
            pltpu.async_copy(
                srcs[s].at[pl.ds(wid * PW + j * CHUNK, CHUNK)], bufs[b], sems[b]
            )

        def wait(b):
            pltpu.make_async_copy(
                srcs[0].at[pl.ds(0, CHUNK)], bufs[b], sems[b]
            ).wait()

        def process(t, b):
            buf = bufs[b]
            hi_off = C * K if tasks[t][0] == NS_IN - 1 else 0

            @plsc.parallel_loop(0, CHUNK // L, unroll=16)
            def ubody(u):
                w = buf[pl.ds(u * L, L)]
                lo = w & lomask
                hi = jax.lax.shift_right_logical(w, 16)
                if hi_off:
                    hi = hi + hi_off
                plsc.addupdate_scatter(hist, [lo], ones16)
                plsc.addupdate_scatter(hist, [hi], ones16)

        start(0, 0)
        for t in range(len(tasks)):
            if t + 1 < len(tasks):
                start(t + 1, (t + 1) % 2)
            wait(t % 2)
            process(t, t % 2)
        pltpu.sync_copy(hist, hist_hbm.at[wid])

    return kern


def _tc_phase2(C):
    TK = 256

    def kern(h_ref, out_ref):
        h = jnp.sum(h_ref[...].astype(jnp.float32), axis=0)
        h2 = h.reshape(1, 2 * C * K)
        hall = jnp.concatenate(
            [h2[:, c * K : (c + 1) * K] for c in range(C)], axis=0
        )
        hfg = jnp.concatenate(
            [h2[:, (C + c) * K : (C + c + 1) * K] for c in range(C)], axis=0
        )
        hbg = hall - hfg
        gts = jnp.sum(hfg, axis=1, keepdims=True)
        jrow = lax.broadcasted_iota(jnp.int32, (K, TK), 0)
        kcol = lax.broadcasted_iota(jnp.int32, (K, TK), 1)
        total = jnp.float32(0.0)
        f0sum = jnp.float32(0.0)
        for tile in range(K // TK):
            k0 = tile * TK
            m1 = jnp.where(jrow + kcol + k0 < K, 1.0, 0.0)
            m2 = jnp.where(jrow >= kcol + k0, 1.0, 0.0)
            a = jnp.dot(hfg, m1, preferred_element_type=jnp.float32)
            b = jnp.dot(hbg, m2, preferred_element_type=jnp.float32)
            denom = gts + b
            F = jnp.where(
                denom > 0.0,
                1.0 - (gts - a) / jnp.where(denom > 0.0, denom, 1.0),
                0.0,
            )
            total = total + jnp.sum(F)
            if tile == 0:
                f0sum = jnp.sum(F[:, 0:1])
        out_ref[0, 0] = (total - 0.5 * f0sum) / jnp.float32(C * K)

    return kern


def kernel(probas, labels):
    P, C = probas.shape
    probasT = probas.T
    streams = _tc_bucketize(P, C)(probasT, labels)
    hist = _sc_scatter(P, C)(*streams)
    out = pl.pallas_call(
        _tc_phase2(C),
        out_shape=jax.ShapeDtypeStruct((1, 1), jnp.float32),
        out_specs=pl.BlockSpec(memory_space=pltpu.SMEM),
    )(hist)
    return out[0, 0]
